# baseline (speedup 1.0000x reference)
import functools

import jax
import jax.numpy as jnp
from jax import lax
from jax.experimental import pallas as pl
from jax.experimental.pallas import tpu as pltpu
from jax.experimental.pallas import tpu_sc as plsc

B = 16384
N_CONT = 13
N_CAT = 26
V = 100001
D = 32
N_OUT = N_CONT + N_CAT
EPS = 1e-5

NW = 32
BPT = B // NW
NB = 32
NCHUNK = BPT // NB
NBT = 1024


def _stats_body(x_ref, g_ref, beta_ref, o_ref):
    x = x_ref[...]
    mean = jnp.mean(x, axis=0)
    var = jnp.mean((x - mean) ** 2, axis=0)
    scale = g_ref[...] * lax.rsqrt(var + EPS)
    shift = beta_ref[...] - mean * scale
    o_ref[...] = jnp.stack([scale, shift])


def _apply_body(x_ref, ss_ref, w_ref, b_ref, o_ref):
    xn = x_ref[...] * ss_ref[0][None, :] + ss_ref[1][None, :]
    xc = w_ref[...][None] * xn[:, :, None] + b_ref[...][None]
    o_ref[...] = xc.reshape(NBT * N_CONT, D)


_mesh = plsc.VectorSubcoreMesh(core_axis_name="c", subcore_axis_name="s")


@functools.partial(
    pl.kernel,
    mesh=_mesh,
    compiler_params=pltpu.CompilerParams(
        needs_layout_passes=False, use_tc_tiling_on_sc=False),
    out_type=jax.ShapeDtypeStruct((B * N_OUT, D), jnp.float32),
    scratch_types=[
        pltpu.VMEM((NB * N_CAT + 16,), jnp.int32),
        pltpu.VMEM((NB * 32,), jnp.int32),
        pltpu.VMEM((NB * N_OUT, D), jnp.float32),
        pltpu.SemaphoreType.DMA,
        pltpu.SemaphoreType.DMA,
    ],
)
def _sc_embed(xc_hbm, cat_hbm, tab_hbm, out_hbm,
              raw_v, idx1, block, sem, csem):
    wid = lax.axis_index("s") * 2 + lax.axis_index("c")
    lanes = lax.iota(jnp.int32, 16)
    ov0 = lanes * V
    f1 = lanes + 16
    ov1 = jnp.where(f1 < N_CAT, f1, 0) * V

    def chunk_body(ci, carry):
        base = wid * BPT + ci * NB
        pltpu.sync_copy(cat_hbm.at[pl.ds(base * N_CAT, NB * N_CAT)],
                        raw_v.at[pl.ds(0, NB * N_CAT)])

        for i in range(NB):
            g0 = plsc.load_gather(raw_v, [lanes + (i * N_CAT)])
            g1 = plsc.load_gather(raw_v, [lanes + (i * N_CAT + 16)])
            plsc.store_scatter(idx1, [lanes + (i * 32)], g0 + ov0)
            plsc.store_scatter(idx1, [lanes + (i * 32 + 16)], g1 + ov1)

        cps = []
        for i in range(NB):
            cps.append(pltpu.async_copy(
                tab_hbm.at[idx1.at[pl.ds(i * 32, N_CAT)]],
                block.at[pl.ds(i * N_OUT + N_CONT, N_CAT)],
                sem))
            cps.append(pltpu.async_copy(
                xc_hbm.at[pl.ds((base + i) * N_CONT, N_CONT)],
                block.at[pl.ds(i * N_OUT, N_CONT)],
                csem))
        for cp in cps:
            cp.wait()

        pltpu.sync_copy(block, out_hbm.at[pl.ds(base * N_OUT, NB * N_OUT)])
        return carry

    lax.fori_loop(0, NCHUNK, chunk_body, 0)


def kernel(x, categorical, gamma, beta, W, b, tables):
    ss = pl.pallas_call(
        _stats_body,
        out_shape=jax.ShapeDtypeStruct((2, N_CONT), jnp.float32),
    )(x, gamma, beta)
    xc = pl.pallas_call(
        _apply_body,
        grid=(B // NBT,),
        in_specs=[
            pl.BlockSpec((NBT, N_CONT), lambda i: (i, 0)),
            pl.BlockSpec((2, N_CONT), lambda i: (0, 0)),
            pl.BlockSpec((N_CONT, D), lambda i: (0, 0)),
            pl.BlockSpec((N_CONT, D), lambda i: (0, 0)),
        ],
        out_specs=pl.BlockSpec((NBT * N_CONT, D), lambda i: (i, 0)),
        out_shape=jax.ShapeDtypeStruct((B * N_CONT, D), jnp.float32),
    )(x, ss, W, b)
    cf = categorical.astype(jnp.int32).reshape(-1)
    tf = tables.reshape(N_CAT * V, D)
    out = _sc_embed(xc, cf, tf)
    return out.reshape(B, N_OUT, D)
